# initial kernel scaffold (unmeasured)
import jax
import jax.numpy as jnp
from jax import lax
from jax.experimental import pallas as pl
from jax.experimental.pallas import tpu as pltpu

N_DEV = 4
N_STEP = 4


def kernel(x, w_mat, scale_x, scale_w):
    m_full, kp = x.shape
    _, n_full = w_mat.shape
    mp = m_full // N_DEV
    nb = n_full // N_STEP

    def body(x_ref, w_ref, sx_ref, sw_ref, out_ref,
             w_all, x_all, w_snd, w_rcv, x_snd, x_rcv):
        step = pl.program_id(0)
        i = lax.axis_index("i")
        left = lax.rem(i + N_DEV - 1, N_DEV)
        right = lax.rem(i + 1, N_DEV)
        diag = lax.rem(i + 2, N_DEV)

        barrier = pltpu.get_barrier_semaphore()
        for nbr in (left, right, diag):
            pl.semaphore_signal(
                barrier, inc=1,
                device_id=(nbr,), device_id_type=pl.DeviceIdType.MESH,
            )
        pl.semaphore_wait(barrier, 3)

        x_rdmas = []
        for r in range(1, N_DEV):
            dst = lax.rem(i + r, N_DEV)
            x_rdmas.append(pltpu.make_async_remote_copy(
                src_ref=x_ref.at[pl.ds(dst * mp, mp), :],
                dst_ref=x_all.at[r - 1],
                send_sem=x_snd.at[r - 1],
                recv_sem=x_rcv.at[r - 1],
                device_id=(dst,),
                device_id_type=pl.DeviceIdType.MESH,
            ))

        @pl.when(step == 0)
        def _():
            for rdma in x_rdmas:
                rdma.start()

        for h in range(1, N_DEV):
            src = w_ref if h == 1 else w_all.at[h - 2]
            rdma = pltpu.make_async_remote_copy(
                src_ref=src,
                dst_ref=w_all.at[h - 1],
                send_sem=w_snd.at[h - 1],
                recv_sem=w_rcv.at[h - 1],
                device_id=(right,),
                device_id_type=pl.DeviceIdType.MESH,
            )
            rdma.start()
            rdma.wait()

        @pl.when(step == 0)
        def _():
            for rdma in x_rdmas:
                rdma.wait()

        acc = jnp.dot(x_ref[pl.ds(i * mp, mp), :], w_ref[...],
                      preferred_element_type=jnp.float32)
        for h in range(1, N_DEV):
            acc += jnp.dot(x_all[h - 1], w_all[h - 1],
                           preferred_element_type=jnp.float32)

        y = acc * (sx_ref[0] * sw_ref[0])
        out_ref[...] = y * (1.0 / (1.0 + jnp.exp(-y)))

    return pl.pallas_call(
        body,
        grid=(N_STEP,),
        out_shape=jax.ShapeDtypeStruct((mp, n_full), jnp.float32),
        in_specs=[
            pl.BlockSpec((m_full, kp), lambda s: (0, 0)),
            pl.BlockSpec((kp, nb), lambda s: (0, s)),
            pl.BlockSpec(memory_space=pltpu.SMEM),
            pl.BlockSpec(memory_space=pltpu.SMEM),
        ],
        out_specs=pl.BlockSpec((mp, nb), lambda s: (0, s)),
        scratch_shapes=[
            pltpu.VMEM((N_DEV - 1, kp, nb), w_mat.dtype),
            pltpu.VMEM((N_DEV - 1, mp, kp), x.dtype),
            pltpu.SemaphoreType.DMA((N_DEV - 1,)),
            pltpu.SemaphoreType.DMA((N_DEV - 1,)),
            pltpu.SemaphoreType.DMA((N_DEV - 1,)),
            pltpu.SemaphoreType.DMA((N_DEV - 1,)),
        ],
        compiler_params=pltpu.CompilerParams(
            collective_id=0,
            dimension_semantics=("arbitrary",),
        ),
    )(x, w_mat, scale_x, scale_w)


# baseline (device time: 415422 ns/iter reference)
import jax
import jax.numpy as jnp
from jax import lax
from jax.experimental import pallas as pl
from jax.experimental.pallas import tpu as pltpu

N_DEV = 4
N_STEP = 4


def kernel(x, w_mat, scale_x, scale_w):
    x = x.astype(jnp.float8_e4m3fn)
    w_mat = w_mat.astype(jnp.float8_e4m3fn)
    m_full, kp = x.shape
    _, n_full = w_mat.shape
    mp = m_full // N_DEV
    nb = n_full // N_STEP

    def body(x_ref, w_ref, sx_ref, sw_ref, out_ref,
             w_all, x_all, w_snd, w_rcv, x_snd, x_rcv):
        step = pl.program_id(0)
        i = lax.axis_index("i")
        left = lax.rem(i + N_DEV - 1, N_DEV)
        right = lax.rem(i + 1, N_DEV)
        diag = lax.rem(i + 2, N_DEV)

        barrier = pltpu.get_barrier_semaphore()
        for nbr in (left, right, diag):
            pl.semaphore_signal(
                barrier, inc=1,
                device_id=(nbr,), device_id_type=pl.DeviceIdType.MESH,
            )
        pl.semaphore_wait(barrier, 3)

        x_rdmas = []
        for r in range(1, N_DEV):
            dst = lax.rem(i + r, N_DEV)
            x_rdmas.append(pltpu.make_async_remote_copy(
                src_ref=x_ref.at[pl.ds(dst * mp, mp), :],
                dst_ref=x_all.at[r - 1],
                send_sem=x_snd.at[r - 1],
                recv_sem=x_rcv.at[r - 1],
                device_id=(dst,),
                device_id_type=pl.DeviceIdType.MESH,
            ))

        @pl.when(step == 0)
        def _():
            for rdma in x_rdmas:
                rdma.start()

        for h in range(1, N_DEV):
            src = w_ref if h == 1 else w_all.at[h - 2]
            rdma = pltpu.make_async_remote_copy(
                src_ref=src,
                dst_ref=w_all.at[h - 1],
                send_sem=w_snd.at[h - 1],
                recv_sem=w_rcv.at[h - 1],
                device_id=(right,),
                device_id_type=pl.DeviceIdType.MESH,
            )
            rdma.start()
            rdma.wait()

        @pl.when(step == 0)
        def _():
            for rdma in x_rdmas:
                rdma.wait()

        acc = jnp.dot(x_ref[pl.ds(i * mp, mp), :], w_ref[...],
                      preferred_element_type=jnp.float32)
        for h in range(1, N_DEV):
            acc += jnp.dot(x_all[h - 1], w_all[h - 1],
                           preferred_element_type=jnp.float32)

        y = acc * (sx_ref[0] * sw_ref[0])
        out_ref[...] = y * (1.0 / (1.0 + jnp.exp(-y)))

    return pl.pallas_call(
        body,
        grid=(N_STEP,),
        out_shape=jax.ShapeDtypeStruct((mp, n_full), jnp.float32),
        in_specs=[
            pl.BlockSpec((m_full, kp), lambda s: (0, 0)),
            pl.BlockSpec((kp, nb), lambda s: (0, s)),
            pl.BlockSpec(memory_space=pltpu.SMEM),
            pl.BlockSpec(memory_space=pltpu.SMEM),
        ],
        out_specs=pl.BlockSpec((mp, nb), lambda s: (0, s)),
        scratch_shapes=[
            pltpu.VMEM((N_DEV - 1, kp, nb), w_mat.dtype),
            pltpu.VMEM((N_DEV - 1, mp, kp), x.dtype),
            pltpu.SemaphoreType.DMA((N_DEV - 1,)),
            pltpu.SemaphoreType.DMA((N_DEV - 1,)),
            pltpu.SemaphoreType.DMA((N_DEV - 1,)),
            pltpu.SemaphoreType.DMA((N_DEV - 1,)),
        ],
        compiler_params=pltpu.CompilerParams(
            collective_id=0,
            dimension_semantics=("arbitrary",),
            vmem_limit_bytes=60 * 1024 * 1024,
        ),
    )(x, w_mat, scale_x, scale_w)


# device time: 301229 ns/iter; 1.3791x vs baseline; 1.3791x over previous
import jax
import jax.numpy as jnp
from jax import lax
from jax.experimental import pallas as pl
from jax.experimental.pallas import tpu as pltpu

N_DEV = 4
N_STEP = 4


def kernel(x, w_mat, scale_x, scale_w):
    x = x.astype(jnp.float8_e4m3fn)
    w_mat = w_mat.astype(jnp.float8_e4m3fn)
    m_full, kp = x.shape
    _, n_full = w_mat.shape
    mp = m_full // N_DEV
    nb = n_full // N_STEP

    def body(x_ref, w_ref, sx_ref, sw_ref, out_ref,
             w_all, x_all, w_snd, w_rcv, x_snd, x_rcv):
        step = pl.program_id(0)
        i = lax.axis_index("i")
        left = lax.rem(i + N_DEV - 1, N_DEV)
        right = lax.rem(i + 1, N_DEV)
        diag = lax.rem(i + 2, N_DEV)

        barrier = pltpu.get_barrier_semaphore()
        for nbr in (left, right, diag):
            pl.semaphore_signal(
                barrier, inc=1,
                device_id=(nbr,), device_id_type=pl.DeviceIdType.MESH,
            )
        pl.semaphore_wait(barrier, 3)

        x_rdmas = []
        for r in range(1, N_DEV):
            dst = lax.rem(i + r, N_DEV)
            x_rdmas.append(pltpu.make_async_remote_copy(
                src_ref=x_ref.at[pl.ds(dst * mp, mp), :],
                dst_ref=x_all.at[r - 1],
                send_sem=x_snd.at[r - 1],
                recv_sem=x_rcv.at[r - 1],
                device_id=(dst,),
                device_id_type=pl.DeviceIdType.MESH,
            ))

        w_rdmas = []
        for r in range(1, N_DEV):
            dst = lax.rem(i + r, N_DEV)
            w_rdmas.append(pltpu.make_async_remote_copy(
                src_ref=w_ref,
                dst_ref=w_all.at[r - 1],
                send_sem=w_snd.at[r - 1],
                recv_sem=w_rcv.at[r - 1],
                device_id=(dst,),
                device_id_type=pl.DeviceIdType.MESH,
            ))
        for rdma in w_rdmas:
            rdma.start()

        @pl.when(step == 0)
        def _():
            for rdma in x_rdmas:
                rdma.start()

        acc = jnp.dot(x_ref[pl.ds(i * mp, mp), :], w_ref[...],
                      preferred_element_type=jnp.float32)

        @pl.when(step == 0)
        def _():
            for rdma in x_rdmas:
                rdma.wait()

        for r in (1, 3, 2):
            w_rdmas[r - 1].wait()
            acc += jnp.dot(x_all[r - 1], w_all[r - 1],
                           preferred_element_type=jnp.float32)

        y = acc * (sx_ref[0] * sw_ref[0])
        out_ref[...] = y * (1.0 / (1.0 + jnp.exp(-y)))

    return pl.pallas_call(
        body,
        grid=(N_STEP,),
        out_shape=jax.ShapeDtypeStruct((mp, n_full), jnp.float32),
        in_specs=[
            pl.BlockSpec((m_full, kp), lambda s: (0, 0)),
            pl.BlockSpec((kp, nb), lambda s: (0, s)),
            pl.BlockSpec(memory_space=pltpu.SMEM),
            pl.BlockSpec(memory_space=pltpu.SMEM),
        ],
        out_specs=pl.BlockSpec((mp, nb), lambda s: (0, s)),
        scratch_shapes=[
            pltpu.VMEM((N_DEV - 1, kp, nb), w_mat.dtype),
            pltpu.VMEM((N_DEV - 1, mp, kp), x.dtype),
            pltpu.SemaphoreType.DMA((N_DEV - 1,)),
            pltpu.SemaphoreType.DMA((N_DEV - 1,)),
            pltpu.SemaphoreType.DMA((N_DEV - 1,)),
            pltpu.SemaphoreType.DMA((N_DEV - 1,)),
        ],
        compiler_params=pltpu.CompilerParams(
            collective_id=0,
            dimension_semantics=("arbitrary",),
            vmem_limit_bytes=60 * 1024 * 1024,
        ),
    )(x, w_mat, scale_x, scale_w)
